# baseline (speedup 1.0000x reference)
import functools

import jax
import jax.numpy as jnp
import numpy as np
from jax.experimental import pallas as pl
from jax.experimental.pallas import tpu as pltpu
from jax.sharding import Mesh, PartitionSpec as P

try:
    from jax.experimental.shard_map import shard_map as _shard_map
except ImportError:
    _shard_map = jax.shard_map

_D_MODEL = 1024
_D_STATE = 128
_D_CONV = 4
_D_INNER = 2048
_HEADDIM = 64
_NHEADS = 32
_CONV_DIM = 2304
_D_IN_PROJ = 4384
_Q = 128


def _chunk_body(u_ref, wi_ref, cw_ref, cb_ref, dtb_ref, alog_ref, d_ref,
                nw_ref, wo_ref, e_ref, out_ref, s_ref, tail_ref,
                wi_t_ref, wo_t_ref, *, rev):
    c = pl.program_id(1)
    Q = _Q

    @pl.when(c == 0)
    def _init():
        s_ref[...] = jnp.zeros_like(s_ref)
        tail_ref[...] = jnp.zeros_like(tail_ref)
        wi_t_ref[...] = wi_ref[0].T
        wo_t_ref[...] = wo_ref[0].T

    rows_q = jax.lax.broadcasted_iota(jnp.int32, (Q, Q), 0)
    cols_q = jax.lax.broadcasted_iota(jnp.int32, (Q, Q), 1)

    u_blk = u_ref[0]
    if rev:
        R16 = (rows_q + cols_q == Q - 1).astype(jnp.bfloat16)
        u_blk = jnp.dot(R16, u_blk, preferred_element_type=jnp.float32
                        ).astype(jnp.bfloat16)
    zx = jnp.dot(u_blk, wi_t_ref[...], preferred_element_type=jnp.float32)

    z = zx[:, :_D_INNER]
    xbc_raw = zx[:, _D_INNER:_D_INNER + _CONV_DIM]
    dt_raw = zx[:, _D_INNER + _CONV_DIM:]

    ext = jnp.concatenate([tail_ref[0:3, :], xbc_raw], axis=0)
    cw = cw_ref[0]
    conv = (cw[0:1, :] * ext[0:Q, :] + cw[1:2, :] * ext[1:Q + 1, :]
            + cw[2:3, :] * ext[2:Q + 2, :] + cw[3:4, :] * ext[3:Q + 3, :])
    conv = conv + cb_ref[0]
    tail_ref[0:3, :] = xbc_raw[Q - 3:Q, :]
    xbc = conv * jax.nn.sigmoid(conv)

    x = xbc[:, :_D_INNER]
    Bm = xbc[:, _D_INNER:_D_INNER + _D_STATE]
    Cm = xbc[:, _D_INNER + _D_STATE:]

    d_arg = dt_raw + dtb_ref[0]
    dm = jnp.minimum(d_arg, 20.0)
    dt = jnp.log(1.0 + jnp.exp(dm)) + jnp.maximum(d_arg - 20.0, 0.0)
    a = dt * (-jnp.exp(alog_ref[0]))

    mask = rows_q >= cols_q

    tri = mask.astype(jnp.bfloat16)
    a_hi = a.astype(jnp.bfloat16)
    a_lo = (a - a_hi.astype(jnp.float32)).astype(jnp.bfloat16)
    dn = (((1,), (0,)), ((), ()))
    cs = (jax.lax.dot_general(tri, a_hi, dn, preferred_element_type=jnp.float32)
          + jax.lax.dot_general(tri, a_lo, dn, preferred_element_type=jnp.float32))
    a_sum = cs[Q - 1:Q, :]

    E = e_ref[...]

    V = jnp.concatenate([dt, dt * jnp.exp(a_sum - cs), jnp.exp(cs),
                         jnp.exp(a_sum), d_ref[0]], axis=0)
    V_hi = V.astype(jnp.bfloat16)
    V_lo = (V - V_hi.astype(jnp.float32)).astype(jnp.bfloat16)
    EX = (jax.lax.dot_general(V_hi, E, dn, preferred_element_type=jnp.float32)
          + jax.lax.dot_general(V_lo, E, dn, preferred_element_type=jnp.float32))

    dtx = EX[0:Q, :] * x
    w_st = EX[Q:2 * Q, :] * x

    S = s_ref[...]
    y_inter = jnp.dot(Cm, S, preferred_element_type=jnp.float32) * EX[2 * Q:3 * Q, :]

    s_ref[...] = (EX[3 * Q:3 * Q + 1, :] * S
                  + jax.lax.dot_general(Bm, w_st, (((0,), (0,)), ((), ())),
                                        preferred_element_type=jnp.float32))

    CB = jax.lax.dot_general(Cm, Bm, (((1,), (1,)), ((), ())),
                             preferred_element_type=jnp.float32)
    csT = cs.T
    pieces = []
    for h in range(_NHEADS):
        diff = jnp.where(mask, cs[:, h:h + 1] - csT[h:h + 1, :], -1e30)
        Mh = CB * jnp.exp(diff)
        pieces.append(jnp.dot(Mh, dtx[:, h * _HEADDIM:(h + 1) * _HEADDIM],
                              preferred_element_type=jnp.float32))
    y_intra = jnp.concatenate(pieces, axis=1)

    y = y_intra + y_inter + EX[3 * Q + 1:3 * Q + 2, :] * x

    y = y * (z * jax.nn.sigmoid(z))
    ms = jnp.mean(y * y, axis=1, keepdims=True)
    y = y * jax.lax.rsqrt(ms + 1e-5) * nw_ref[0]
    y16 = y.astype(jnp.bfloat16)
    if rev:
        y16 = jnp.dot(R16, y16, preferred_element_type=jnp.float32
                      ).astype(jnp.bfloat16)
    out_ref[0] = jnp.dot(y16, wo_t_ref[...], preferred_element_type=jnp.float32)


def _run_dir(u16, wi, cw, cb, dtb, alog, dd, nw, wo, E, rev):
    L = u16.shape[1]
    nchunks = L // _Q
    if rev:
        tsel = lambda b, c: (b, nchunks - 1 - c, 0)
    else:
        tsel = lambda b, c: (b, c, 0)
    wsel3 = lambda b, c: (0, 0, 0)
    return pl.pallas_call(
        functools.partial(_chunk_body, rev=rev),
        grid=(2, nchunks),
        in_specs=[
            pl.BlockSpec((1, _Q, _D_MODEL), tsel),
            pl.BlockSpec((1, _D_IN_PROJ, _D_MODEL), wsel3),
            pl.BlockSpec((1, _D_CONV, _CONV_DIM), wsel3),
            pl.BlockSpec((1, 1, _CONV_DIM), wsel3),
            pl.BlockSpec((1, 1, _NHEADS), wsel3),
            pl.BlockSpec((1, 1, _NHEADS), wsel3),
            pl.BlockSpec((1, 1, _NHEADS), wsel3),
            pl.BlockSpec((1, 1, _D_INNER), wsel3),
            pl.BlockSpec((1, _D_MODEL, _D_INNER), wsel3),
            pl.BlockSpec((_NHEADS, _D_INNER), lambda b, c: (0, 0)),
        ],
        out_specs=pl.BlockSpec((1, _Q, _D_MODEL), tsel),
        out_shape=jax.ShapeDtypeStruct((2, L, _D_MODEL), jnp.float32),
        scratch_shapes=[
            pltpu.VMEM((_D_STATE, _D_INNER), jnp.float32),
            pltpu.VMEM((8, _CONV_DIM), jnp.float32),
            pltpu.VMEM((_D_MODEL, _D_IN_PROJ), jnp.bfloat16),
            pltpu.VMEM((_D_INNER, _D_MODEL), jnp.bfloat16),
        ],
        compiler_params=pltpu.CompilerParams(
            dimension_semantics=("parallel", "arbitrary"),
            vmem_limit_bytes=56 * 1024 * 1024,
        ),
    )(u16, wi, cw, cb, dtb, alog, dd, nw, wo, E)


@jax.jit
def kernel(u, Wi_f, conv_w_f, conv_b_f, dt_bias_f, A_log_f, D_f, norm_w_f,
           Wo_f, Wi_b, conv_w_b, conv_b_b, dt_bias_b, A_log_b, D_b, norm_w_b,
           Wo_b):
    u16 = u.astype(jnp.bfloat16)
    wi = jnp.stack([Wi_f, Wi_b]).astype(jnp.bfloat16)
    wo = jnp.stack([Wo_f, Wo_b]).astype(jnp.bfloat16)
    cw = jnp.stack([conv_w_f[:, 0, :].T, conv_w_b[:, 0, :].T])
    cb = jnp.stack([conv_b_f, conv_b_b])[:, None, :]
    dtb = jnp.stack([dt_bias_f, dt_bias_b])[:, None, :]
    alog = jnp.stack([A_log_f, A_log_b])[:, None, :]
    dd = jnp.stack([D_f, D_b])[:, None, :]
    nw = jnp.stack([norm_w_f, norm_w_b])[:, None, :]

    heads = jnp.arange(_NHEADS, dtype=jnp.int32)[:, None]
    cols = jnp.arange(_D_INNER, dtype=jnp.int32)[None, :]
    E = (cols // _HEADDIM == heads).astype(jnp.bfloat16)

    devs = jax.devices()
    if len(devs) >= 2:
        mesh = Mesh(np.array(devs[:2]), ("d",))

        def _sharded(u16, wi, cw, cb, dtb, alog, dd, nw, wo, E):
            out_local = jax.lax.cond(
                jax.lax.axis_index("d") == 0,
                lambda: _run_dir(u16, wi, cw, cb, dtb, alog, dd, nw, wo, E,
                                 rev=False),
                lambda: _run_dir(u16, wi, cw, cb, dtb, alog, dd, nw, wo, E,
                                 rev=True),
            )
            return jax.lax.psum(out_local, "d") * 0.5

        ws = P("d", None, None)
        return _shard_map(
            _sharded, mesh=mesh,
            in_specs=(P(), ws, ws, ws, ws, ws, ws, ws, ws, P()),
            out_specs=P(), check_rep=False,
        )(u16, wi, cw, cb, dtb, alog, dd, nw, wo, E)

    out_f = _run_dir(u16, wi[0:1], cw[0:1], cb[0:1], dtb[0:1], alog[0:1],
                     dd[0:1], nw[0:1], wo[0:1], E, rev=False)
    out_b = _run_dir(u16, wi[1:2], cw[1:2], cb[1:2], dtb[1:2], alog[1:2],
                     dd[1:2], nw[1:2], wo[1:2], E, rev=True)
    return (out_f + out_b) * 0.5


# speedup vs baseline: 1.6938x; 1.6938x over previous
import functools

import jax
import jax.numpy as jnp
import numpy as np
from jax.experimental import pallas as pl
from jax.experimental.pallas import tpu as pltpu
from jax.sharding import Mesh, PartitionSpec as P

try:
    from jax.experimental.shard_map import shard_map as _shard_map
except ImportError:
    _shard_map = jax.shard_map

_D_MODEL = 1024
_D_STATE = 128
_D_CONV = 4
_D_INNER = 2048
_HEADDIM = 64
_NHEADS = 32
_CONV_DIM = 2304
_D_IN_PROJ = 4384
_Q = 256


def _chunk_body(u_ref, wi_ref, cw_ref, cb_ref, dtb_ref, alog_ref, d_ref,
                nw_ref, wo_ref, e_ref, out_ref, s_ref, tail_ref,
                wi_t_ref, wo_t_ref, *, rev):
    c = pl.program_id(1)
    Q = _Q

    @pl.when(c == 0)
    def _init():
        s_ref[...] = jnp.zeros_like(s_ref)
        tail_ref[...] = jnp.zeros_like(tail_ref)
        wi_t_ref[...] = wi_ref[0].T
        wo_t_ref[...] = wo_ref[0].T

    rows_q = jax.lax.broadcasted_iota(jnp.int32, (Q, Q), 0)
    cols_q = jax.lax.broadcasted_iota(jnp.int32, (Q, Q), 1)

    u_blk = u_ref[0]
    if rev:
        R16 = (rows_q + cols_q == Q - 1).astype(jnp.bfloat16)
        u_blk = jnp.dot(R16, u_blk, preferred_element_type=jnp.float32
                        ).astype(jnp.bfloat16)
    zx = jnp.dot(u_blk, wi_t_ref[...], preferred_element_type=jnp.float32)

    z = zx[:, :_D_INNER]
    xbc_raw = zx[:, _D_INNER:_D_INNER + _CONV_DIM]
    dt_raw = zx[:, _D_INNER + _CONV_DIM:]

    ext = jnp.concatenate([tail_ref[0:3, :], xbc_raw], axis=0)
    cw = cw_ref[0]
    conv = (cw[0:1, :] * ext[0:Q, :] + cw[1:2, :] * ext[1:Q + 1, :]
            + cw[2:3, :] * ext[2:Q + 2, :] + cw[3:4, :] * ext[3:Q + 3, :])
    conv = conv + cb_ref[0]
    tail_ref[0:3, :] = xbc_raw[Q - 3:Q, :]
    xbc = conv * jax.nn.sigmoid(conv)

    x = xbc[:, :_D_INNER]
    Bm = xbc[:, _D_INNER:_D_INNER + _D_STATE]
    Cm = xbc[:, _D_INNER + _D_STATE:]

    d_arg = dt_raw + dtb_ref[0]
    dm = jnp.minimum(d_arg, 20.0)
    dt = jnp.log(1.0 + jnp.exp(dm)) + jnp.maximum(d_arg - 20.0, 0.0)
    a = dt * (-jnp.exp(alog_ref[0]))

    mask = rows_q >= cols_q

    tri = mask.astype(jnp.bfloat16)
    a_hi = a.astype(jnp.bfloat16)
    a_lo = (a - a_hi.astype(jnp.float32)).astype(jnp.bfloat16)
    dn = (((1,), (0,)), ((), ()))
    cs = (jax.lax.dot_general(tri, a_hi, dn, preferred_element_type=jnp.float32)
          + jax.lax.dot_general(tri, a_lo, dn, preferred_element_type=jnp.float32))
    a_sum = cs[Q - 1:Q, :]

    E = e_ref[...]

    V = jnp.concatenate([dt, dt * jnp.exp(a_sum - cs), jnp.exp(cs),
                         jnp.exp(a_sum), d_ref[0]], axis=0)
    V_hi = V.astype(jnp.bfloat16)
    V_lo = (V - V_hi.astype(jnp.float32)).astype(jnp.bfloat16)
    EX = (jax.lax.dot_general(V_hi, E, dn, preferred_element_type=jnp.float32)
          + jax.lax.dot_general(V_lo, E, dn, preferred_element_type=jnp.float32))

    dtx = EX[0:Q, :] * x
    w_st = EX[Q:2 * Q, :] * x

    S = s_ref[...]
    y_inter = jnp.dot(Cm, S, preferred_element_type=jnp.float32) * EX[2 * Q:3 * Q, :]

    s_ref[...] = (EX[3 * Q:3 * Q + 1, :] * S
                  + jax.lax.dot_general(Bm, w_st, (((0,), (0,)), ((), ())),
                                        preferred_element_type=jnp.float32))

    CB = jax.lax.dot_general(Cm, Bm, (((1,), (1,)), ((), ())),
                             preferred_element_type=jnp.float32)
    csT = cs.T
    pieces = []
    for h in range(_NHEADS):
        diff = jnp.where(mask, cs[:, h:h + 1] - csT[h:h + 1, :], -1e30)
        Mh = CB * jnp.exp(diff)
        pieces.append(jnp.dot(Mh, dtx[:, h * _HEADDIM:(h + 1) * _HEADDIM],
                              preferred_element_type=jnp.float32))
    y_intra = jnp.concatenate(pieces, axis=1)

    y = y_intra + y_inter + EX[3 * Q + 1:3 * Q + 2, :] * x

    y = y * (z * jax.nn.sigmoid(z))
    ms = jnp.mean(y * y, axis=1, keepdims=True)
    y = y * jax.lax.rsqrt(ms + 1e-5) * nw_ref[0]
    y16 = y.astype(jnp.bfloat16)
    if rev:
        y16 = jnp.dot(R16, y16, preferred_element_type=jnp.float32
                      ).astype(jnp.bfloat16)
    out_ref[0] = jnp.dot(y16, wo_t_ref[...], preferred_element_type=jnp.float32)


def _run_dir(u16, wi, cw, cb, dtb, alog, dd, nw, wo, E, rev):
    L = u16.shape[1]
    nchunks = L // _Q
    if rev:
        tsel = lambda b, c: (b, nchunks - 1 - c, 0)
    else:
        tsel = lambda b, c: (b, c, 0)
    wsel3 = lambda b, c: (0, 0, 0)
    return pl.pallas_call(
        functools.partial(_chunk_body, rev=rev),
        grid=(2, nchunks),
        in_specs=[
            pl.BlockSpec((1, _Q, _D_MODEL), tsel),
            pl.BlockSpec((1, _D_IN_PROJ, _D_MODEL), wsel3),
            pl.BlockSpec((1, _D_CONV, _CONV_DIM), wsel3),
            pl.BlockSpec((1, 1, _CONV_DIM), wsel3),
            pl.BlockSpec((1, 1, _NHEADS), wsel3),
            pl.BlockSpec((1, 1, _NHEADS), wsel3),
            pl.BlockSpec((1, 1, _NHEADS), wsel3),
            pl.BlockSpec((1, 1, _D_INNER), wsel3),
            pl.BlockSpec((1, _D_MODEL, _D_INNER), wsel3),
            pl.BlockSpec((_NHEADS, _D_INNER), lambda b, c: (0, 0)),
        ],
        out_specs=pl.BlockSpec((1, _Q, _D_MODEL), tsel),
        out_shape=jax.ShapeDtypeStruct((2, L, _D_MODEL), jnp.float32),
        scratch_shapes=[
            pltpu.VMEM((_D_STATE, _D_INNER), jnp.float32),
            pltpu.VMEM((8, _CONV_DIM), jnp.float32),
            pltpu.VMEM((_D_MODEL, _D_IN_PROJ), jnp.bfloat16),
            pltpu.VMEM((_D_INNER, _D_MODEL), jnp.bfloat16),
        ],
        compiler_params=pltpu.CompilerParams(
            dimension_semantics=("parallel", "arbitrary"),
            vmem_limit_bytes=56 * 1024 * 1024,
        ),
    )(u16, wi, cw, cb, dtb, alog, dd, nw, wo, E)


@jax.jit
def kernel(u, Wi_f, conv_w_f, conv_b_f, dt_bias_f, A_log_f, D_f, norm_w_f,
           Wo_f, Wi_b, conv_w_b, conv_b_b, dt_bias_b, A_log_b, D_b, norm_w_b,
           Wo_b):
    u16 = u.astype(jnp.bfloat16)
    wi = jnp.stack([Wi_f, Wi_b]).astype(jnp.bfloat16)
    wo = jnp.stack([Wo_f, Wo_b]).astype(jnp.bfloat16)
    cw = jnp.stack([conv_w_f[:, 0, :].T, conv_w_b[:, 0, :].T])
    cb = jnp.stack([conv_b_f, conv_b_b])[:, None, :]
    dtb = jnp.stack([dt_bias_f, dt_bias_b])[:, None, :]
    alog = jnp.stack([A_log_f, A_log_b])[:, None, :]
    dd = jnp.stack([D_f, D_b])[:, None, :]
    nw = jnp.stack([norm_w_f, norm_w_b])[:, None, :]

    heads = jnp.arange(_NHEADS, dtype=jnp.int32)[:, None]
    cols = jnp.arange(_D_INNER, dtype=jnp.int32)[None, :]
    E = (cols // _HEADDIM == heads).astype(jnp.bfloat16)

    devs = jax.devices()
    if False and len(devs) >= 2:
        mesh = Mesh(np.array(devs[:2]), ("d",))

        def _sharded(u16, wi, cw, cb, dtb, alog, dd, nw, wo, E):
            out_local = jax.lax.cond(
                jax.lax.axis_index("d") == 0,
                lambda: _run_dir(u16, wi, cw, cb, dtb, alog, dd, nw, wo, E,
                                 rev=False),
                lambda: _run_dir(u16, wi, cw, cb, dtb, alog, dd, nw, wo, E,
                                 rev=True),
            )
            return jax.lax.psum(out_local, "d") * 0.5

        ws = P("d", None, None)
        return _shard_map(
            _sharded, mesh=mesh,
            in_specs=(P(), ws, ws, ws, ws, ws, ws, ws, ws, P()),
            out_specs=P(), check_rep=False,
        )(u16, wi, cw, cb, dtb, alog, dd, nw, wo, E)

    out_f = _run_dir(u16, wi[0:1], cw[0:1], cb[0:1], dtb[0:1], alog[0:1],
                     dd[0:1], nw[0:1], wo[0:1], E, rev=False)
    out_b = _run_dir(u16, wi[1:2], cw[1:2], cb[1:2], dtb[1:2], alog[1:2],
                     dd[1:2], nw[1:2], wo[1:2], E, rev=True)
    return (out_f + out_b) * 0.5


# speedup vs baseline: 1.6941x; 1.0002x over previous
import functools

import jax
import jax.numpy as jnp
from jax.experimental import pallas as pl
from jax.experimental.pallas import tpu as pltpu

_D_MODEL = 1024
_D_STATE = 128
_D_CONV = 4
_D_INNER = 2048
_HEADDIM = 64
_NHEADS = 32
_CONV_DIM = 2304
_D_IN_PROJ = 4384
_Q = 256


def _chunk_body(u_ref, wi_ref, cw_ref, cb_ref, dtb_ref, alog_ref, d_ref,
                nw_ref, wo_ref, e_ref, out_ref, s_ref, tail_ref,
                wi_t_ref, wo_t_ref, *, rev):
    c = pl.program_id(1)
    Q = _Q

    @pl.when(c == 0)
    def _init():
        s_ref[...] = jnp.zeros_like(s_ref)
        tail_ref[...] = jnp.zeros_like(tail_ref)
        wi_t_ref[...] = wi_ref[0].T
        wo_t_ref[...] = wo_ref[0].T

    rows_q = jax.lax.broadcasted_iota(jnp.int32, (Q, Q), 0)
    cols_q = jax.lax.broadcasted_iota(jnp.int32, (Q, Q), 1)

    u_blk = u_ref[0]
    if rev:
        R16 = (rows_q + cols_q == Q - 1).astype(jnp.bfloat16)
        u_blk = jnp.dot(R16, u_blk, preferred_element_type=jnp.float32
                        ).astype(jnp.bfloat16)
    zx = jnp.dot(u_blk, wi_t_ref[...], preferred_element_type=jnp.float32)

    z = zx[:, :_D_INNER]
    xbc_raw = zx[:, _D_INNER:_D_INNER + _CONV_DIM]
    dt_raw = zx[:, _D_INNER + _CONV_DIM:]

    ext = jnp.concatenate([tail_ref[0:3, :], xbc_raw], axis=0)
    cw = cw_ref[0]
    conv = (cw[0:1, :] * ext[0:Q, :] + cw[1:2, :] * ext[1:Q + 1, :]
            + cw[2:3, :] * ext[2:Q + 2, :] + cw[3:4, :] * ext[3:Q + 3, :])
    conv = conv + cb_ref[0]
    tail_ref[0:3, :] = xbc_raw[Q - 3:Q, :]
    xbc = conv * jax.nn.sigmoid(conv)

    x = xbc[:, :_D_INNER]
    Bm = xbc[:, _D_INNER:_D_INNER + _D_STATE]
    Cm = xbc[:, _D_INNER + _D_STATE:]

    d_arg = dt_raw + dtb_ref[0]
    dm = jnp.minimum(d_arg, 20.0)
    dt = jnp.log(1.0 + jnp.exp(dm)) + jnp.maximum(d_arg - 20.0, 0.0)
    a = dt * (-jnp.exp(alog_ref[0]))

    mask = rows_q >= cols_q

    tri = mask.astype(jnp.bfloat16)
    a_hi = a.astype(jnp.bfloat16)
    a_lo = (a - a_hi.astype(jnp.float32)).astype(jnp.bfloat16)
    dn = (((1,), (0,)), ((), ()))
    cs = (jax.lax.dot_general(tri, a_hi, dn, preferred_element_type=jnp.float32)
          + jax.lax.dot_general(tri, a_lo, dn, preferred_element_type=jnp.float32))
    a_sum = cs[Q - 1:Q, :]

    E = e_ref[...]

    V = jnp.concatenate([dt, dt * jnp.exp(a_sum - cs), jnp.exp(cs),
                         jnp.exp(a_sum), d_ref[0]], axis=0)
    V_hi = V.astype(jnp.bfloat16)
    V_lo = (V - V_hi.astype(jnp.float32)).astype(jnp.bfloat16)
    EX = (jax.lax.dot_general(V_hi, E, dn, preferred_element_type=jnp.float32)
          + jax.lax.dot_general(V_lo, E, dn, preferred_element_type=jnp.float32))

    dtx = EX[0:Q, :] * x
    w_st = EX[Q:2 * Q, :] * x

    S = s_ref[...]
    y_inter = jnp.dot(Cm, S, preferred_element_type=jnp.float32) * EX[2 * Q:3 * Q, :]

    s_ref[...] = (EX[3 * Q:3 * Q + 1, :] * S
                  + jax.lax.dot_general(Bm, w_st, (((0,), (0,)), ((), ())),
                                        preferred_element_type=jnp.float32))

    CB = jax.lax.dot_general(Cm, Bm, (((1,), (1,)), ((), ())),
                             preferred_element_type=jnp.float32)
    csT = cs.T
    pieces = []
    for h in range(_NHEADS):
        diff = jnp.where(mask, cs[:, h:h + 1] - csT[h:h + 1, :], -1e30)
        Mh = CB * jnp.exp(diff)
        pieces.append(jnp.dot(Mh, dtx[:, h * _HEADDIM:(h + 1) * _HEADDIM],
                              preferred_element_type=jnp.float32))
    y_intra = jnp.concatenate(pieces, axis=1)

    y = y_intra + y_inter + EX[3 * Q + 1:3 * Q + 2, :] * x

    y = y * (z * jax.nn.sigmoid(z))
    ms = jnp.mean(y * y, axis=1, keepdims=True)
    y = y * jax.lax.rsqrt(ms + 1e-5) * nw_ref[0]
    y16 = y.astype(jnp.bfloat16)
    if rev:
        y16 = jnp.dot(R16, y16, preferred_element_type=jnp.float32
                      ).astype(jnp.bfloat16)
    out_ref[0] = jnp.dot(y16, wo_t_ref[...], preferred_element_type=jnp.float32)


def _run_dir(u16, wi, cw, cb, dtb, alog, dd, nw, wo, E, rev):
    L = u16.shape[1]
    nchunks = L // _Q
    if rev:
        tsel = lambda b, c: (b, nchunks - 1 - c, 0)
    else:
        tsel = lambda b, c: (b, c, 0)
    wsel3 = lambda b, c: (0, 0, 0)
    return pl.pallas_call(
        functools.partial(_chunk_body, rev=rev),
        grid=(2, nchunks),
        in_specs=[
            pl.BlockSpec((1, _Q, _D_MODEL), tsel),
            pl.BlockSpec((1, _D_IN_PROJ, _D_MODEL), wsel3),
            pl.BlockSpec((1, _D_CONV, _CONV_DIM), wsel3),
            pl.BlockSpec((1, 1, _CONV_DIM), wsel3),
            pl.BlockSpec((1, 1, _NHEADS), wsel3),
            pl.BlockSpec((1, 1, _NHEADS), wsel3),
            pl.BlockSpec((1, 1, _NHEADS), wsel3),
            pl.BlockSpec((1, 1, _D_INNER), wsel3),
            pl.BlockSpec((1, _D_MODEL, _D_INNER), wsel3),
            pl.BlockSpec((_NHEADS, _D_INNER), lambda b, c: (0, 0)),
        ],
        out_specs=pl.BlockSpec((1, _Q, _D_MODEL), tsel),
        out_shape=jax.ShapeDtypeStruct((2, L, _D_MODEL), jnp.float32),
        scratch_shapes=[
            pltpu.VMEM((_D_STATE, _D_INNER), jnp.float32),
            pltpu.VMEM((8, _CONV_DIM), jnp.float32),
            pltpu.VMEM((_D_MODEL, _D_IN_PROJ), jnp.bfloat16),
            pltpu.VMEM((_D_INNER, _D_MODEL), jnp.bfloat16),
        ],
        compiler_params=pltpu.CompilerParams(
            dimension_semantics=("parallel", "arbitrary"),
            vmem_limit_bytes=56 * 1024 * 1024,
        ),
    )(u16, wi, cw, cb, dtb, alog, dd, nw, wo, E)


@jax.jit
def kernel(u, Wi_f, conv_w_f, conv_b_f, dt_bias_f, A_log_f, D_f, norm_w_f,
           Wo_f, Wi_b, conv_w_b, conv_b_b, dt_bias_b, A_log_b, D_b, norm_w_b,
           Wo_b):
    u16 = u.astype(jnp.bfloat16)
    wi = jnp.stack([Wi_f, Wi_b]).astype(jnp.bfloat16)
    wo = jnp.stack([Wo_f, Wo_b]).astype(jnp.bfloat16)
    cw = jnp.stack([conv_w_f[:, 0, :].T, conv_w_b[:, 0, :].T])
    cb = jnp.stack([conv_b_f, conv_b_b])[:, None, :]
    dtb = jnp.stack([dt_bias_f, dt_bias_b])[:, None, :]
    alog = jnp.stack([A_log_f, A_log_b])[:, None, :]
    dd = jnp.stack([D_f, D_b])[:, None, :]
    nw = jnp.stack([norm_w_f, norm_w_b])[:, None, :]

    heads = jnp.arange(_NHEADS, dtype=jnp.int32)[:, None]
    cols = jnp.arange(_D_INNER, dtype=jnp.int32)[None, :]
    E = (cols // _HEADDIM == heads).astype(jnp.bfloat16)

    out_f = _run_dir(u16, wi[0:1], cw[0:1], cb[0:1], dtb[0:1], alog[0:1],
                     dd[0:1], nw[0:1], wo[0:1], E, rev=False)
    out_b = _run_dir(u16, wi[1:2], cw[1:2], cb[1:2], dtb[1:2], alog[1:2],
                     dd[1:2], nw[1:2], wo[1:2], E, rev=True)
    return (out_f + out_b) * 0.5
